# Initial kernel scaffold; baseline (speedup 1.0000x reference)
import jax
import jax.numpy as jnp
from jax.experimental import pallas as pl


def kernel(pred_logits, pred_boxes, tgt_labels, tgt_boxes):
    raise NotImplementedError("write your pallas kernel here")


# speedup vs baseline: 4.0770x; 4.0770x over previous
import jax
import jax.numpy as jnp
from jax.experimental import pallas as pl
from jax.experimental.pallas import tpu as pltpu

_COST_CLASS = 1.0
_COST_BBOX = 5.0
_COST_GIOU = 2.0

_TQ = 456


def _cost_kernel(logits_ref, boxes_ref, labels_ref, tbt_ref, out_ref):
    logits = logits_ref[0]
    mx = jnp.max(logits, axis=-1, keepdims=True)
    e = jnp.exp(logits - mx)
    prob = e / jnp.sum(e, axis=-1, keepdims=True)

    labels = labels_ref[...]
    c_dim = logits.shape[-1]
    t_dim = labels.shape[-1]
    iota_c = jax.lax.broadcasted_iota(jnp.int32, (c_dim, t_dim), 0)
    onehot = (iota_c == labels).astype(jnp.float32)
    prob_at = jnp.dot(prob, onehot, preferred_element_type=jnp.float32,
                      precision=jax.lax.Precision.HIGHEST)

    qb = boxes_ref[0]
    qcx, qcy = qb[:, 0:1], qb[:, 1:2]
    qw, qh = qb[:, 2:3], qb[:, 3:4]
    tcx, tcy = tbt_ref[0:1, :], tbt_ref[1:2, :]
    tw, th = tbt_ref[2:3, :], tbt_ref[3:4, :]

    cost_bbox = (jnp.abs(qcx - tcx) + jnp.abs(qcy - tcy)
                 + jnp.abs(qw - tw) + jnp.abs(qh - th))

    qx1, qx2 = qcx - 0.5 * qw, qcx + 0.5 * qw
    qy1, qy2 = qcy - 0.5 * qh, qcy + 0.5 * qh
    tx1, tx2 = tcx - 0.5 * tw, tcx + 0.5 * tw
    ty1, ty2 = tcy - 0.5 * th, tcy + 0.5 * th

    iw = jnp.clip(jnp.minimum(qx2, tx2) - jnp.maximum(qx1, tx1), 0.0)
    ih = jnp.clip(jnp.minimum(qy2, ty2) - jnp.maximum(qy1, ty1), 0.0)
    inter = iw * ih
    area_q = (qx2 - qx1) * (qy2 - qy1)
    area_t = (tx2 - tx1) * (ty2 - ty1)
    union = area_q + area_t - inter
    ew = jnp.clip(jnp.maximum(qx2, tx2) - jnp.minimum(qx1, tx1), 0.0)
    eh = jnp.clip(jnp.maximum(qy2, ty2) - jnp.minimum(qy1, ty1), 0.0)
    area_e = ew * eh
    giou = inter / union - (area_e - union) / area_e

    out_ref[0] = (_COST_BBOX * cost_bbox - _COST_CLASS * prob_at
                  - _COST_GIOU * giou)


def kernel(pred_logits, pred_boxes, tgt_labels, tgt_boxes):
    b_dim, q_dim, c_dim = pred_logits.shape
    t_dim = tgt_labels.shape[0]
    labels2d = tgt_labels.astype(jnp.int32).reshape(1, t_dim)
    tbt = tgt_boxes.T
    q_tiles = (q_dim + _TQ - 1) // _TQ
    return pl.pallas_call(
        _cost_kernel,
        out_shape=jax.ShapeDtypeStruct((b_dim, q_dim, t_dim), jnp.float32),
        grid=(b_dim, q_tiles),
        in_specs=[
            pl.BlockSpec((1, _TQ, c_dim), lambda b, q: (b, q, 0)),
            pl.BlockSpec((1, _TQ, 4), lambda b, q: (b, q, 0)),
            pl.BlockSpec((1, t_dim), lambda b, q: (0, 0)),
            pl.BlockSpec((4, t_dim), lambda b, q: (0, 0)),
        ],
        out_specs=pl.BlockSpec((1, _TQ, t_dim), lambda b, q: (b, q, 0)),
        compiler_params=pltpu.CompilerParams(
            dimension_semantics=("parallel", "arbitrary"),
            vmem_limit_bytes=56 * 1024 * 1024,
        ),
        name="hungarian_cost",
    )(pred_logits, pred_boxes, labels2d, tbt)


# speedup vs baseline: 4.6631x; 1.1438x over previous
import jax
import jax.numpy as jnp
from jax.experimental import pallas as pl
from jax.experimental.pallas import tpu as pltpu

_COST_CLASS = 1.0
_COST_BBOX = 5.0
_COST_GIOU = 2.0

_TQ = 456


def _cost_kernel(logits_ref, boxes_ref, labels_ref, tbt_ref, out_ref):
    logits = logits_ref[0]
    mx = jnp.max(logits, axis=-1, keepdims=True)
    e = jnp.exp(logits - mx)
    prob = e / jnp.sum(e, axis=-1, keepdims=True)

    labels = labels_ref[...]
    c_dim = logits.shape[-1]
    t_dim = labels.shape[-1]
    iota_c = jax.lax.broadcasted_iota(jnp.int32, (c_dim, t_dim), 0)
    onehot = (iota_c == labels).astype(jnp.float32)
    prob_at = jnp.dot(prob, onehot, preferred_element_type=jnp.float32)

    qb = boxes_ref[0]
    qcx, qcy = qb[:, 0:1], qb[:, 1:2]
    qw, qh = qb[:, 2:3], qb[:, 3:4]
    tcx, tcy = tbt_ref[0:1, :], tbt_ref[1:2, :]
    tw, th = tbt_ref[2:3, :], tbt_ref[3:4, :]

    cost_bbox = (jnp.abs(qcx - tcx) + jnp.abs(qcy - tcy)
                 + jnp.abs(qw - tw) + jnp.abs(qh - th))

    qx1, qx2 = qcx - 0.5 * qw, qcx + 0.5 * qw
    qy1, qy2 = qcy - 0.5 * qh, qcy + 0.5 * qh
    tx1, tx2 = tcx - 0.5 * tw, tcx + 0.5 * tw
    ty1, ty2 = tcy - 0.5 * th, tcy + 0.5 * th

    dx = jnp.minimum(qx2, tx2) - jnp.maximum(qx1, tx1)
    dy = jnp.minimum(qy2, ty2) - jnp.maximum(qy1, ty1)
    inter = jnp.maximum(dx, 0.0) * jnp.maximum(dy, 0.0)
    area_q = qw * qh
    area_t = tw * th
    union = (area_q + area_t) - inter
    area_e = ((qw + tw) - dx) * ((qh + th) - dy)
    out_ref[0] = ((_COST_BBOX * cost_bbox - _COST_CLASS * prob_at
                   + _COST_GIOU)
                  - _COST_GIOU * (inter / union)
                  - _COST_GIOU * (union / area_e))


def kernel(pred_logits, pred_boxes, tgt_labels, tgt_boxes):
    b_dim, q_dim, c_dim = pred_logits.shape
    t_dim = tgt_labels.shape[0]
    labels2d = tgt_labels.astype(jnp.int32).reshape(1, t_dim)
    tbt = tgt_boxes.T
    q_tiles = (q_dim + _TQ - 1) // _TQ
    return pl.pallas_call(
        _cost_kernel,
        out_shape=jax.ShapeDtypeStruct((b_dim, q_dim, t_dim), jnp.float32),
        grid=(b_dim, q_tiles),
        in_specs=[
            pl.BlockSpec((1, _TQ, c_dim), lambda b, q: (b, q, 0)),
            pl.BlockSpec((1, _TQ, 4), lambda b, q: (b, q, 0)),
            pl.BlockSpec((1, t_dim), lambda b, q: (0, 0)),
            pl.BlockSpec((4, t_dim), lambda b, q: (0, 0)),
        ],
        out_specs=pl.BlockSpec((1, _TQ, t_dim), lambda b, q: (b, q, 0)),
        compiler_params=pltpu.CompilerParams(
            dimension_semantics=("parallel", "arbitrary"),
            vmem_limit_bytes=56 * 1024 * 1024,
        ),
        name="hungarian_cost",
    )(pred_logits, pred_boxes, labels2d, tbt)
